# baseline (speedup 1.0000x reference)
import functools

import jax
import jax.numpy as jnp
from jax import lax
from jax.experimental import pallas as pl
from jax.experimental.pallas import tpu as pltpu
from jax.experimental.pallas import tpu_sc as plsc

N_NODES = 10000
D = 128
NC = 2
NS = 16
NW = NC * NS
EPB = 128
N_ACC = 10048
CH_STRIDE = 632
CH_TAIL = N_ACC - 15 * CH_STRIDE
B0 = 32
B1 = 128
DEG_W = 128
ROWS = 400
_MESH = dict(core_axis_name="c", subcore_axis_name="s")


def _copy_chunk(s, src, dst):
    b = s * CH_STRIDE

    @pl.when(s < NS - 1)
    def _():
        pltpu.sync_copy(src.at[pl.ds(b, CH_STRIDE)], dst.at[pl.ds(b, CH_STRIDE)])

    @pl.when(s == NS - 1)
    def _():
        pltpu.sync_copy(src.at[pl.ds(b, CH_TAIL)], dst.at[pl.ds(b, CH_TAIL)])


def _deg_call(n_blk, dst_r, ones_hbm, zeros_hbm):
    @functools.partial(
        pl.kernel,
        out_type=jax.ShapeDtypeStruct((NC, N_ACC, DEG_W), jnp.float32),
        mesh=plsc.VectorSubcoreMesh(**_MESH),
        scratch_types=[
            pltpu.VMEM((n_blk, EPB), jnp.int32),
            pltpu.VMEM((EPB, DEG_W), jnp.float32),
            pltpu.VMEM_SHARED((N_ACC, DEG_W), jnp.float32),
        ],
    )
    def deg_kernel(dst_hbm, ones_h, zeros_h, deg_hbm, idx_v, ones_v, acc_sh):
        c = lax.axis_index("c")
        s = lax.axis_index("s")
        wid = c * NS + s
        _copy_chunk(s, zeros_h, acc_sh)
        pltpu.sync_copy(dst_hbm.at[wid], idx_v)
        pltpu.sync_copy(ones_h, ones_v)
        plsc.subcore_barrier()

        @pl.loop(0, n_blk)
        def _(j):
            pltpu.sync_copy(ones_v, acc_sh.at[idx_v.at[j]], add=True)

        plsc.subcore_barrier()
        _copy_chunk(s, acc_sh, deg_hbm.at[c])

    return deg_kernel(dst_r, ones_hbm, zeros_hbm)


def _unpack_block(pk, j, t, row):
    for k in range(EPB // 16):
        v = pk[j, pl.ds(k * 16, 16)]
        t[row, pl.ds(k * 16, 16)] = lax.bitwise_and(v, jnp.int32(16383))
        t[row + 1, pl.ds(k * 16, 16)] = lax.shift_right_logical(v, jnp.int32(14))


def _scatter_call(y, pk_flat, zeros_hbm):
    @functools.partial(
        pl.kernel,
        out_type=jax.ShapeDtypeStruct((NC, N_ACC, D), jnp.float32),
        mesh=plsc.VectorSubcoreMesh(**_MESH),
        scratch_types=[
            pltpu.VMEM((B1, EPB), jnp.int32),
            pltpu.VMEM((4, EPB), jnp.int32),
            pltpu.VMEM((EPB, D), jnp.float32),
            pltpu.VMEM((EPB, D), jnp.float32),
            pltpu.VMEM_SHARED((N_ACC, D), jnp.float32),
            pltpu.SemaphoreType.DMA,
            pltpu.SemaphoreType.DMA,
        ],
    )
    def scat_kernel(y_hbm, pk_hbm, zeros_h, out_hbm,
                    pk, ix, g0, g1, acc_sh, gsem0, gsem1):
        c = lax.axis_index("c")
        s = lax.axis_index("s")
        start = jnp.where(c == 0, s * B0, NS * B0 + s * B1)
        nb = jnp.where(c == 0, B0, B1)
        _copy_chunk(s, zeros_h, acc_sh)
        pltpu.sync_copy(pk_hbm.at[pl.ds(start, B1)], pk)
        plsc.subcore_barrier()

        _unpack_block(pk, 0, ix, 0)
        pltpu.async_copy(y_hbm.at[ix.at[0]], g0, gsem0)
        _unpack_block(pk, 1, ix, 2)
        pltpu.async_copy(y_hbm.at[ix.at[2]], g1, gsem1)

        @pl.loop(0, nb - 2, step=2)
        def _(j):
            pltpu.make_async_copy(y_hbm.at[ix.at[0]], g0, gsem0).wait()
            pltpu.sync_copy(g0, acc_sh.at[ix.at[1]], add=True)
            _unpack_block(pk, j + 2, ix, 0)
            pltpu.async_copy(y_hbm.at[ix.at[0]], g0, gsem0)
            pltpu.make_async_copy(y_hbm.at[ix.at[2]], g1, gsem1).wait()
            pltpu.sync_copy(g1, acc_sh.at[ix.at[3]], add=True)
            _unpack_block(pk, j + 3, ix, 2)
            pltpu.async_copy(y_hbm.at[ix.at[2]], g1, gsem1)

        pltpu.make_async_copy(y_hbm.at[ix.at[0]], g0, gsem0).wait()
        pltpu.sync_copy(g0, acc_sh.at[ix.at[1]], add=True)
        pltpu.make_async_copy(y_hbm.at[ix.at[2]], g1, gsem1).wait()
        pltpu.sync_copy(g1, acc_sh.at[ix.at[3]], add=True)

        plsc.subcore_barrier()
        _copy_chunk(s, acc_sh, out_hbm.at[c])

    return scat_kernel(y, pk_flat, zeros_hbm)


def _pre_call(x, W1, deg2):
    def body(x_ref, w_ref, deg_ref, y_ref, dinv_ref):
        d = deg_ref[0, :, 0:1] + deg_ref[1, :, 0:1] + 1.0
        dinv = lax.rsqrt(d)
        xw = jnp.dot(x_ref[...], w_ref[...], preferred_element_type=jnp.float32)
        y_ref[...] = xw * dinv
        dinv_ref[...] = jnp.broadcast_to(dinv, (ROWS, D))

    return pl.pallas_call(
        body,
        grid=(N_NODES // ROWS,),
        in_specs=[
            pl.BlockSpec((ROWS, D), lambda i: (i, 0)),
            pl.BlockSpec((D, D), lambda i: (0, 0)),
            pl.BlockSpec((NC, ROWS, DEG_W), lambda i: (0, i, 0)),
        ],
        out_specs=[
            pl.BlockSpec((ROWS, D), lambda i: (i, 0)),
            pl.BlockSpec((ROWS, D), lambda i: (i, 0)),
        ],
        out_shape=[
            jax.ShapeDtypeStruct((N_NODES, D), jnp.float32),
            jax.ShapeDtypeStruct((N_NODES, D), jnp.float32),
        ],
    )(x, W1, deg2)


def _mid_call(acc, y, dinv, b, h_prev, W_next):
    def body(acc_ref, y_ref, dinv_ref, b_ref, hp_ref, w_ref, h_ref, yn_ref):
        conv = dinv_ref[...] * (acc_ref[0] + acc_ref[1] + y_ref[...]) + b_ref[...]
        h = jnp.maximum(conv + hp_ref[...], 0.0)
        h_ref[...] = h
        hw = jnp.dot(h, w_ref[...], preferred_element_type=jnp.float32)
        yn_ref[...] = hw * dinv_ref[...]

    return pl.pallas_call(
        body,
        grid=(N_NODES // ROWS,),
        in_specs=[
            pl.BlockSpec((NC, ROWS, D), lambda i: (0, i, 0)),
            pl.BlockSpec((ROWS, D), lambda i: (i, 0)),
            pl.BlockSpec((ROWS, D), lambda i: (i, 0)),
            pl.BlockSpec((1, D), lambda i: (0, 0)),
            pl.BlockSpec((ROWS, D), lambda i: (i, 0)),
            pl.BlockSpec((D, D), lambda i: (0, 0)),
        ],
        out_specs=[
            pl.BlockSpec((ROWS, D), lambda i: (i, 0)),
            pl.BlockSpec((ROWS, D), lambda i: (i, 0)),
        ],
        out_shape=[
            jax.ShapeDtypeStruct((N_NODES, D), jnp.float32),
            jax.ShapeDtypeStruct((N_NODES, D), jnp.float32),
        ],
    )(acc, y, dinv, b, h_prev, W_next)


def kernel(x, edge_index, W1, b1, W2, b2, W3, b3):
    ei = edge_index.astype(jnp.int32)
    src, dst = ei[0], ei[1]
    e = src.shape[0]
    e_pad = NS * (B0 + B1) * EPB
    assert e <= e_pad
    n_blk = e_pad // (NW * EPB)
    pad = e_pad - e
    src_p = jnp.concatenate([src, jnp.zeros((pad,), jnp.int32)])
    dst_p = jnp.concatenate([dst, jnp.full((pad,), N_NODES, jnp.int32)])
    dst_r = dst_p.reshape(NW, n_blk, EPB)
    pk_flat = (dst_p * 16384 + src_p).reshape(e_pad // EPB, EPB)

    ones16 = jnp.ones((EPB, DEG_W), jnp.float32)
    zeros_deg = jnp.zeros((N_ACC, DEG_W), jnp.float32)
    zeros_acc = jnp.zeros((N_ACC, D), jnp.float32)

    deg2 = _deg_call(n_blk, dst_r, ones16, zeros_deg)
    y1, dinv = _pre_call(x, W1, deg2)

    w_stack = jnp.stack([W2, W3, W3])
    b_stack = jnp.stack([b1, b2, b3]).reshape(3, 1, D)

    def step(carry, wb):
        h_prev, y = carry
        w_next, b = wb
        acc = _scatter_call(y, pk_flat, zeros_acc)
        h, y_next = _mid_call(acc, y, dinv, b, h_prev, w_next)
        return (h, y_next), None

    (h3, _), _ = lax.scan(step, (x, y1), (w_stack, b_stack))
    return h3


# speedup vs baseline: 1.2661x; 1.2661x over previous
import functools

import jax
import jax.numpy as jnp
from jax import lax
from jax.experimental import pallas as pl
from jax.experimental.pallas import tpu as pltpu
from jax.experimental.pallas import tpu_sc as plsc

N_NODES = 10000
D = 128
NC = 2
NS = 16
NW = NC * NS
EPB = 128
N_ACC = 10048
CH_STRIDE = 632
CH_TAIL = N_ACC - 15 * CH_STRIDE
B0 = 128
B1 = 32
BMAX = max(B0, B1)
PK_ROWS = NS * B0 + (NS - 1) * B1 + BMAX
DEG_W = 128
ROWS = 400
_MESH = dict(core_axis_name="c", subcore_axis_name="s")


def _copy_chunk(s, src, dst):
    b = s * CH_STRIDE

    @pl.when(s < NS - 1)
    def _():
        pltpu.sync_copy(src.at[pl.ds(b, CH_STRIDE)], dst.at[pl.ds(b, CH_STRIDE)])

    @pl.when(s == NS - 1)
    def _():
        pltpu.sync_copy(src.at[pl.ds(b, CH_TAIL)], dst.at[pl.ds(b, CH_TAIL)])


def _deg_call(n_blk, dst_r, ones_hbm, zeros_hbm):
    @functools.partial(
        pl.kernel,
        out_type=jax.ShapeDtypeStruct((NC, N_ACC, DEG_W), jnp.float32),
        mesh=plsc.VectorSubcoreMesh(**_MESH),
        scratch_types=[
            pltpu.VMEM((n_blk, EPB), jnp.int32),
            pltpu.VMEM((EPB, DEG_W), jnp.float32),
            pltpu.VMEM_SHARED((N_ACC, DEG_W), jnp.float32),
        ],
    )
    def deg_kernel(dst_hbm, ones_h, zeros_h, deg_hbm, idx_v, ones_v, acc_sh):
        c = lax.axis_index("c")
        s = lax.axis_index("s")
        wid = c * NS + s
        _copy_chunk(s, zeros_h, acc_sh)
        pltpu.sync_copy(dst_hbm.at[wid], idx_v)
        pltpu.sync_copy(ones_h, ones_v)
        plsc.subcore_barrier()

        @pl.loop(0, n_blk)
        def _(j):
            pltpu.sync_copy(ones_v, acc_sh.at[idx_v.at[j]], add=True)

        plsc.subcore_barrier()
        _copy_chunk(s, acc_sh, deg_hbm.at[c])

    return deg_kernel(dst_r, ones_hbm, zeros_hbm)


def _unpack_block(pk, j, t, row):
    for k in range(EPB // 16):
        v = pk[j, pl.ds(k * 16, 16)]
        t[row, pl.ds(k * 16, 16)] = lax.bitwise_and(v, jnp.int32(16383))
        t[row + 1, pl.ds(k * 16, 16)] = lax.shift_right_logical(v, jnp.int32(14))


def _scatter_call(y, pk_flat, zeros_hbm):
    @functools.partial(
        pl.kernel,
        out_type=jax.ShapeDtypeStruct((NC, N_ACC, D), jnp.float32),
        mesh=plsc.VectorSubcoreMesh(**_MESH),
        scratch_types=[
            pltpu.VMEM((BMAX, EPB), jnp.int32),
            pltpu.VMEM((4, EPB), jnp.int32),
            pltpu.VMEM((EPB, D), jnp.float32),
            pltpu.VMEM((EPB, D), jnp.float32),
            pltpu.VMEM_SHARED((N_ACC, D), jnp.float32),
            pltpu.SemaphoreType.DMA,
            pltpu.SemaphoreType.DMA,
        ],
    )
    def scat_kernel(y_hbm, pk_hbm, zeros_h, out_hbm,
                    pk, ix, g0, g1, acc_sh, gsem0, gsem1):
        c = lax.axis_index("c")
        s = lax.axis_index("s")
        start = jnp.where(c == 0, s * B0, NS * B0 + s * B1)
        nb = jnp.where(c == 0, B0, B1)
        _copy_chunk(s, zeros_h, acc_sh)
        pltpu.sync_copy(pk_hbm.at[pl.ds(start, BMAX)], pk)
        plsc.subcore_barrier()

        _unpack_block(pk, 0, ix, 0)
        pltpu.async_copy(y_hbm.at[ix.at[0]], g0, gsem0)
        _unpack_block(pk, 1, ix, 2)
        pltpu.async_copy(y_hbm.at[ix.at[2]], g1, gsem1)

        @pl.loop(0, nb - 2, step=2)
        def _(j):
            pltpu.make_async_copy(y_hbm.at[ix.at[0]], g0, gsem0).wait()
            pltpu.sync_copy(g0, acc_sh.at[ix.at[1]], add=True)
            _unpack_block(pk, j + 2, ix, 0)
            pltpu.async_copy(y_hbm.at[ix.at[0]], g0, gsem0)
            pltpu.make_async_copy(y_hbm.at[ix.at[2]], g1, gsem1).wait()
            pltpu.sync_copy(g1, acc_sh.at[ix.at[3]], add=True)
            _unpack_block(pk, j + 3, ix, 2)
            pltpu.async_copy(y_hbm.at[ix.at[2]], g1, gsem1)

        pltpu.make_async_copy(y_hbm.at[ix.at[0]], g0, gsem0).wait()
        pltpu.sync_copy(g0, acc_sh.at[ix.at[1]], add=True)
        pltpu.make_async_copy(y_hbm.at[ix.at[2]], g1, gsem1).wait()
        pltpu.sync_copy(g1, acc_sh.at[ix.at[3]], add=True)

        plsc.subcore_barrier()
        _copy_chunk(s, acc_sh, out_hbm.at[c])

    return scat_kernel(y, pk_flat, zeros_hbm)


def _pre_call(x, W1, deg2):
    def body(x_ref, w_ref, deg_ref, y_ref, dinv_ref):
        d = deg_ref[0, :, 0:1] + deg_ref[1, :, 0:1] + 1.0
        dinv = lax.rsqrt(d)
        xw = jnp.dot(x_ref[...], w_ref[...], preferred_element_type=jnp.float32)
        y_ref[...] = xw * dinv
        dinv_ref[...] = jnp.broadcast_to(dinv, (ROWS, D))

    return pl.pallas_call(
        body,
        grid=(N_NODES // ROWS,),
        in_specs=[
            pl.BlockSpec((ROWS, D), lambda i: (i, 0)),
            pl.BlockSpec((D, D), lambda i: (0, 0)),
            pl.BlockSpec((NC, ROWS, DEG_W), lambda i: (0, i, 0)),
        ],
        out_specs=[
            pl.BlockSpec((ROWS, D), lambda i: (i, 0)),
            pl.BlockSpec((ROWS, D), lambda i: (i, 0)),
        ],
        out_shape=[
            jax.ShapeDtypeStruct((N_NODES, D), jnp.float32),
            jax.ShapeDtypeStruct((N_NODES, D), jnp.float32),
        ],
    )(x, W1, deg2)


def _mid_call(acc, y, dinv, b, h_prev, W_next):
    def body(acc_ref, y_ref, dinv_ref, b_ref, hp_ref, w_ref, h_ref, yn_ref):
        conv = dinv_ref[...] * (acc_ref[0] + acc_ref[1] + y_ref[...]) + b_ref[...]
        h = jnp.maximum(conv + hp_ref[...], 0.0)
        h_ref[...] = h
        hw = jnp.dot(h, w_ref[...], preferred_element_type=jnp.float32)
        yn_ref[...] = hw * dinv_ref[...]

    return pl.pallas_call(
        body,
        grid=(N_NODES // ROWS,),
        in_specs=[
            pl.BlockSpec((NC, ROWS, D), lambda i: (0, i, 0)),
            pl.BlockSpec((ROWS, D), lambda i: (i, 0)),
            pl.BlockSpec((ROWS, D), lambda i: (i, 0)),
            pl.BlockSpec((1, D), lambda i: (0, 0)),
            pl.BlockSpec((ROWS, D), lambda i: (i, 0)),
            pl.BlockSpec((D, D), lambda i: (0, 0)),
        ],
        out_specs=[
            pl.BlockSpec((ROWS, D), lambda i: (i, 0)),
            pl.BlockSpec((ROWS, D), lambda i: (i, 0)),
        ],
        out_shape=[
            jax.ShapeDtypeStruct((N_NODES, D), jnp.float32),
            jax.ShapeDtypeStruct((N_NODES, D), jnp.float32),
        ],
    )(acc, y, dinv, b, h_prev, W_next)


def kernel(x, edge_index, W1, b1, W2, b2, W3, b3):
    ei = edge_index.astype(jnp.int32)
    src, dst = ei[0], ei[1]
    e = src.shape[0]
    e_pad = NS * (B0 + B1) * EPB
    assert e <= e_pad
    n_blk = e_pad // (NW * EPB)
    pad = e_pad - e
    src_p = jnp.concatenate([src, jnp.zeros((pad,), jnp.int32)])
    dst_p = jnp.concatenate([dst, jnp.full((pad,), N_NODES, jnp.int32)])
    dst_r = dst_p.reshape(NW, n_blk, EPB)
    pk_flat = (dst_p * 16384 + src_p).reshape(e_pad // EPB, EPB)
    pk_flat = jnp.concatenate(
        [pk_flat, jnp.zeros((PK_ROWS - e_pad // EPB, EPB), jnp.int32)])

    ones16 = jnp.ones((EPB, DEG_W), jnp.float32)
    zeros_deg = jnp.zeros((N_ACC, DEG_W), jnp.float32)
    zeros_acc = jnp.zeros((N_ACC, D), jnp.float32)

    deg2 = _deg_call(n_blk, dst_r, ones16, zeros_deg)
    y1, dinv = _pre_call(x, W1, deg2)

    w_stack = jnp.stack([W2, W3, W3])
    b_stack = jnp.stack([b1, b2, b3]).reshape(3, 1, D)

    def step(carry, wb):
        h_prev, y = carry
        w_next, b = wb
        acc = _scatter_call(y, pk_flat, zeros_acc)
        h, y_next = _mid_call(acc, y, dinv, b, h_prev, w_next)
        return (h, y_next), None

    (h3, _), _ = lax.scan(step, (x, y1), (w_stack, b_stack))
    return h3
